# Initial kernel scaffold; baseline (speedup 1.0000x reference)
import jax
import jax.numpy as jnp
from jax.experimental import pallas as pl


def kernel(x, edge_index, w, attn):
    raise NotImplementedError("write your pallas kernel here")


# speedup vs baseline: 4.9575x; 4.9575x over previous
import functools

import jax
import jax.numpy as jnp
from jax import lax
from jax.experimental import pallas as pl
from jax.experimental.pallas import tpu as pltpu
from jax.experimental.pallas import tpu_sc as plsc

N_HEAD = 4
N_NODES = 10000
N_EDGES = 320000
D = 128

NPAD = 10240
NT = 16
NC = 2
EPT = N_EDGES // NT
CH = 80
NCHUNK = EPT // CH
RPT = NPAD // NT
BN = 400
BF = 512


def _scores_body(x_ref, cs_ref, cd_ref, os_ref, od_ref):
    xb = x_ref[...]
    os_ref[...] = jnp.dot(xb, cs_ref[...], preferred_element_type=jnp.float32)
    od_ref[...] = jnp.dot(xb, cd_ref[...], preferred_element_type=jnp.float32)


def _tc_scores(x, csrc, cdst):
    return pl.pallas_call(
        _scores_body,
        grid=(N_NODES // BN,),
        in_specs=[
            pl.BlockSpec((BN, D), lambda i: (i, 0)),
            pl.BlockSpec((D, N_HEAD), lambda i: (0, 0)),
            pl.BlockSpec((D, N_HEAD), lambda i: (0, 0)),
        ],
        out_specs=[
            pl.BlockSpec((BN, N_HEAD), lambda i: (i, 0)),
            pl.BlockSpec((BN, N_HEAD), lambda i: (i, 0)),
        ],
        out_shape=[
            jax.ShapeDtypeStruct((N_NODES, N_HEAD), jnp.float32),
            jax.ShapeDtypeStruct((N_NODES, N_HEAD), jnp.float32),
        ],
    )(x, csrc, cdst)


def _sc_body(x_hbm, src_hbm, dst_hbm, pk_hbm, p_out, rsp_out,
             P_sh, pk_v, src_v, dst_v, rows_v, rs_v, sem):
    c = lax.axis_index("c")
    s = lax.axis_index("s")
    ebase = s * EPT
    rbase = s * RPT
    zeros16 = jnp.zeros((16,), jnp.float32)

    for p in range(2):
        head = p * NC + c

        def _zero_rows(i, _):
            for kk in range(D // 16):
                rows_v[i, pl.ds(kk * 16, 16)] = zeros16
            return 0

        lax.fori_loop(0, CH, _zero_rows, 0)
        for rep in range(RPT // CH):
            pltpu.sync_copy(rows_v, P_sh.at[pl.ds(rbase + rep * CH, CH)])

        def _zero_rs(i, _):
            rs_v[pl.ds(i * 16, 16)] = zeros16
            return 0

        lax.fori_loop(0, NPAD // 16, _zero_rs, 0)

        pltpu.sync_copy(pk_hbm.at[pl.ds(head * NPAD, NPAD)], pk_v)
        plsc.subcore_barrier()

        def _chunk(k, _):
            off = ebase + k * CH
            pltpu.sync_copy(src_hbm.at[pl.ds(off, CH)], src_v)
            pltpu.sync_copy(dst_hbm.at[pl.ds(off, CH)], dst_v)
            pltpu.async_copy(x_hbm.at[dst_v], rows_v, sem).wait()
            for g in range(CH // 16):
                si = src_v[pl.ds(g * 16, 16)]
                di = dst_v[pl.ds(g * 16, 16)]
                gs = plsc.load_gather(pk_v, [si])
                gd = plsc.load_gather(pk_v, [di])
                z = (plsc.bitcast(gs << 16, jnp.float32)
                     + plsc.bitcast(gd & jnp.int32(-65536), jnp.float32))
                ee = jnp.exp(jnp.where(z >= 0, -z, -0.2 * z))
                plsc.addupdate_scatter(rs_v, [si], ee)
                for lane in range(16):
                    e = g * 16 + lane
                    bc = ee.at[jnp.full((16,), lane, jnp.int32)].get(
                        mode="promise_in_bounds")
                    for kk in range(D // 16):
                        rows_v[e, pl.ds(kk * 16, 16)] = (
                            rows_v[e, pl.ds(kk * 16, 16)] * bc)
            pltpu.sync_copy(rows_v, P_sh.at[src_v], add=True)
            return 0

        lax.fori_loop(0, NCHUNK, _chunk, 0)
        plsc.subcore_barrier()

        pltpu.sync_copy(
            rs_v, rsp_out.at[pl.ds((head * NT + s) * NPAD, NPAD)])
        pltpu.sync_copy(P_sh.at[pl.ds(rbase, RPT)],
                        p_out.at[pl.ds(head * NPAD + rbase, RPT)])
        plsc.subcore_barrier()


def _sc_main(x, src, dst, pk):
    mesh = plsc.VectorSubcoreMesh(core_axis_name="c", subcore_axis_name="s")
    kern = functools.partial(
        pl.kernel,
        mesh=mesh,
        compiler_params=pltpu.CompilerParams(needs_layout_passes=False),
        out_type=[
            jax.ShapeDtypeStruct((N_HEAD * NPAD, D), jnp.float32),
            jax.ShapeDtypeStruct((N_HEAD * NT * NPAD,), jnp.float32),
        ],
        scratch_types=[
            pltpu.VMEM_SHARED((NPAD, D), jnp.float32),
            pltpu.VMEM((NPAD,), jnp.int32),
            pltpu.VMEM((CH,), jnp.int32),
            pltpu.VMEM((CH,), jnp.int32),
            pltpu.VMEM((CH, D), jnp.float32),
            pltpu.VMEM((NPAD,), jnp.float32),
            pltpu.SemaphoreType.DMA,
        ],
    )(_sc_body)
    return kern(x, src, dst, pk)


def _finish_body(p_ref, rsp_ref, w_ref, o_ref):
    rs = jnp.sum(rsp_ref[...], axis=1).reshape(1, BF, 1)
    o_ref[...] = p_ref[...] * w_ref[...] / rs


def _tc_finish(p_acc, rsp, wv3):
    return pl.pallas_call(
        _finish_body,
        grid=(N_HEAD, NPAD // BF),
        in_specs=[
            pl.BlockSpec((1, BF, D), lambda h, i: (h, i, 0)),
            pl.BlockSpec((1, NT, BF), lambda h, i: (h, 0, i)),
            pl.BlockSpec((1, 1, D), lambda h, i: (h, 0, 0)),
        ],
        out_specs=pl.BlockSpec((1, BF, D), lambda h, i: (h, i, 0)),
        out_shape=jax.ShapeDtypeStruct((N_HEAD, NPAD, D), jnp.float32),
    )(p_acc, rsp, wv3)


def kernel(x, edge_index, w, attn):
    src = edge_index[0]
    dst = edge_index[1]
    wv = w[:, 0, :]
    csrc = (wv * attn[:, :D, 0]).T
    cdst = (wv * attn[:, D:, 0]).T
    asrc, adst = _tc_scores(x, csrc, cdst)
    bs = lax.bitcast_convert_type(
        asrc.astype(jnp.bfloat16), jnp.uint16).astype(jnp.uint32)
    bd = lax.bitcast_convert_type(
        adst.astype(jnp.bfloat16), jnp.uint16).astype(jnp.uint32)
    pk = (bs | (bd << 16)).astype(jnp.int32).T
    pk = jnp.pad(pk, ((0, 0), (0, NPAD - N_NODES))).reshape(-1)
    p_acc, rsp = _sc_main(x, src, dst, pk)
    out = _tc_finish(p_acc.reshape(N_HEAD, NPAD, D),
                     rsp.reshape(N_HEAD, NT, NPAD),
                     wv.reshape(N_HEAD, 1, D))
    return out[:, :N_NODES, :]


# speedup vs baseline: 6.4488x; 1.3008x over previous
import functools

import jax
import jax.numpy as jnp
from jax import lax
from jax.experimental import pallas as pl
from jax.experimental.pallas import tpu as pltpu
from jax.experimental.pallas import tpu_sc as plsc

N_HEAD = 4
N_NODES = 10000
N_EDGES = 320000
D = 128

NPAD = 10240
NT = 16
NC = 2
EPT = N_EDGES // NT
CH = 80
NCHUNK = EPT // CH
RPT = NPAD // NT
BN = 400
BF = 512


def _scores_body(x_ref, cs_ref, cd_ref, os_ref, od_ref):
    xb = x_ref[...]
    os_ref[...] = jnp.dot(xb, cs_ref[...], preferred_element_type=jnp.float32)
    od_ref[...] = jnp.dot(xb, cd_ref[...], preferred_element_type=jnp.float32)


def _tc_scores(x, csrc, cdst):
    return pl.pallas_call(
        _scores_body,
        grid=(N_NODES // BN,),
        in_specs=[
            pl.BlockSpec((BN, D), lambda i: (i, 0)),
            pl.BlockSpec((D, N_HEAD), lambda i: (0, 0)),
            pl.BlockSpec((D, N_HEAD), lambda i: (0, 0)),
        ],
        out_specs=[
            pl.BlockSpec((BN, N_HEAD), lambda i: (i, 0)),
            pl.BlockSpec((BN, N_HEAD), lambda i: (i, 0)),
        ],
        out_shape=[
            jax.ShapeDtypeStruct((N_NODES, N_HEAD), jnp.float32),
            jax.ShapeDtypeStruct((N_NODES, N_HEAD), jnp.float32),
        ],
    )(x, csrc, cdst)


def _sc_body(x_hbm, src_hbm, dst_hbm, pk_hbm, p_out, rsp_out,
             P_sh, pk_v, s0_v, d0_v, s1_v, d1_v, rows0, rows1, rs_v,
             semg0, semg1, sems0, sems1):
    c = lax.axis_index("c")
    s = lax.axis_index("s")
    ebase = s * EPT
    rbase = s * RPT
    zeros16 = jnp.zeros((16,), jnp.float32)
    zeros16i = jnp.zeros((16,), jnp.int32)

    def _compute(sv, dv, rows):
        for g in range(CH // 16):
            si = sv[pl.ds(g * 16, 16)]
            di = dv[pl.ds(g * 16, 16)]
            gs = plsc.load_gather(pk_v, [si])
            gd = plsc.load_gather(pk_v, [di])
            z = (plsc.bitcast(gs << 16, jnp.float32)
                 + plsc.bitcast(gd & jnp.int32(-65536), jnp.float32))
            ee = jnp.exp(jnp.where(z >= 0, -z, -0.2 * z))
            plsc.addupdate_scatter(rs_v, [si], ee)
            for lane in range(16):
                e = g * 16 + lane
                bc = ee.at[jnp.full((16,), lane, jnp.int32)].get(
                    mode="promise_in_bounds")
                for kk in range(D // 16):
                    rows[e, pl.ds(kk * 16, 16)] = (
                        rows[e, pl.ds(kk * 16, 16)] * bc)

    for p in range(2):
        head = p * NC + c

        def _zero_rows(i, _):
            for kk in range(D // 16):
                rows0[i, pl.ds(kk * 16, 16)] = zeros16
                rows1[i, pl.ds(kk * 16, 16)] = zeros16
            return 0

        lax.fori_loop(0, CH, _zero_rows, 0)

        def _zero_idx(i, _):
            s1_v[pl.ds(i * 16, 16)] = zeros16i
            return 0

        lax.fori_loop(0, CH // 16, _zero_idx, 0)
        for rep in range(RPT // CH):
            pltpu.sync_copy(rows0, P_sh.at[pl.ds(rbase + rep * CH, CH)])

        def _zero_rs(i, _):
            rs_v[pl.ds(i * 16, 16)] = zeros16
            return 0

        lax.fori_loop(0, NPAD // 16, _zero_rs, 0)

        pltpu.sync_copy(pk_hbm.at[pl.ds(head * NPAD, NPAD)], pk_v)
        plsc.subcore_barrier()

        pltpu.sync_copy(src_hbm.at[pl.ds(ebase, CH)], s0_v)
        pltpu.sync_copy(dst_hbm.at[pl.ds(ebase, CH)], d0_v)
        pltpu.async_copy(x_hbm.at[d0_v], rows0, semg0)
        pltpu.async_copy(rows1, P_sh.at[s1_v], sems1, add=True)

        def _body(j, _):
            off1 = ebase + (2 * j + 1) * CH
            pltpu.make_async_copy(rows1, P_sh.at[s1_v], sems1).wait()
            pltpu.sync_copy(src_hbm.at[pl.ds(off1, CH)], s1_v)
            pltpu.sync_copy(dst_hbm.at[pl.ds(off1, CH)], d1_v)
            pltpu.async_copy(x_hbm.at[d1_v], rows1, semg1)

            pltpu.make_async_copy(x_hbm.at[d0_v], rows0, semg0).wait()
            _compute(s0_v, d0_v, rows0)
            pltpu.async_copy(rows0, P_sh.at[s0_v], sems0, add=True)

            pltpu.make_async_copy(x_hbm.at[d1_v], rows1, semg1).wait()
            _compute(s1_v, d1_v, rows1)
            pltpu.async_copy(rows1, P_sh.at[s1_v], sems1, add=True)

            @pl.when(j < NCHUNK // 2 - 1)
            def _():
                off2 = ebase + (2 * j + 2) * CH
                pltpu.make_async_copy(rows0, P_sh.at[s0_v], sems0).wait()
                pltpu.sync_copy(src_hbm.at[pl.ds(off2, CH)], s0_v)
                pltpu.sync_copy(dst_hbm.at[pl.ds(off2, CH)], d0_v)
                pltpu.async_copy(x_hbm.at[d0_v], rows0, semg0)

            return 0

        lax.fori_loop(0, NCHUNK // 2, _body, 0)
        pltpu.make_async_copy(rows0, P_sh.at[s0_v], sems0).wait()
        pltpu.make_async_copy(rows1, P_sh.at[s1_v], sems1).wait()
        plsc.subcore_barrier()

        pltpu.sync_copy(
            rs_v, rsp_out.at[pl.ds((head * NT + s) * NPAD, NPAD)])
        pltpu.sync_copy(P_sh.at[pl.ds(rbase, RPT)],
                        p_out.at[pl.ds(head * NPAD + rbase, RPT)])
        plsc.subcore_barrier()


def _sc_main(x, src, dst, pk):
    mesh = plsc.VectorSubcoreMesh(core_axis_name="c", subcore_axis_name="s")
    kern = functools.partial(
        pl.kernel,
        mesh=mesh,
        compiler_params=pltpu.CompilerParams(needs_layout_passes=False),
        out_type=[
            jax.ShapeDtypeStruct((N_HEAD * NPAD, D), jnp.float32),
            jax.ShapeDtypeStruct((N_HEAD * NT * NPAD,), jnp.float32),
        ],
        scratch_types=[
            pltpu.VMEM_SHARED((NPAD, D), jnp.float32),
            pltpu.VMEM((NPAD,), jnp.int32),
            pltpu.VMEM((CH,), jnp.int32),
            pltpu.VMEM((CH,), jnp.int32),
            pltpu.VMEM((CH,), jnp.int32),
            pltpu.VMEM((CH,), jnp.int32),
            pltpu.VMEM((CH, D), jnp.float32),
            pltpu.VMEM((CH, D), jnp.float32),
            pltpu.VMEM((NPAD,), jnp.float32),
            pltpu.SemaphoreType.DMA,
            pltpu.SemaphoreType.DMA,
            pltpu.SemaphoreType.DMA,
            pltpu.SemaphoreType.DMA,
        ],
    )(_sc_body)
    return kern(x, src, dst, pk)


def _finish_body(p_ref, rsp_ref, w_ref, o_ref):
    rs = jnp.sum(rsp_ref[...], axis=1).reshape(1, BF, 1)
    o_ref[...] = p_ref[...] * w_ref[...] / rs


def _tc_finish(p_acc, rsp, wv3):
    return pl.pallas_call(
        _finish_body,
        grid=(N_HEAD, NPAD // BF),
        in_specs=[
            pl.BlockSpec((1, BF, D), lambda h, i: (h, i, 0)),
            pl.BlockSpec((1, NT, BF), lambda h, i: (h, 0, i)),
            pl.BlockSpec((1, 1, D), lambda h, i: (h, 0, 0)),
        ],
        out_specs=pl.BlockSpec((1, BF, D), lambda h, i: (h, i, 0)),
        out_shape=jax.ShapeDtypeStruct((N_HEAD, NPAD, D), jnp.float32),
    )(p_acc, rsp, wv3)


def kernel(x, edge_index, w, attn):
    src = edge_index[0]
    dst = edge_index[1]
    wv = w[:, 0, :]
    csrc = (wv * attn[:, :D, 0]).T
    cdst = (wv * attn[:, D:, 0]).T
    asrc, adst = _tc_scores(x, csrc, cdst)
    bs = lax.bitcast_convert_type(
        asrc.astype(jnp.bfloat16), jnp.uint16).astype(jnp.uint32)
    bd = lax.bitcast_convert_type(
        adst.astype(jnp.bfloat16), jnp.uint16).astype(jnp.uint32)
    pk = (bs | (bd << 16)).astype(jnp.int32).T
    pk = jnp.pad(pk, ((0, 0), (0, NPAD - N_NODES))).reshape(-1)
    p_acc, rsp = _sc_main(x, src, dst, pk)
    out = _tc_finish(p_acc.reshape(N_HEAD, NPAD, D),
                     rsp.reshape(N_HEAD, NT, NPAD),
                     wv.reshape(N_HEAD, 1, D))
    return out[:, :N_NODES, :]


# speedup vs baseline: 7.7519x; 1.2021x over previous
import functools

import jax
import jax.numpy as jnp
from jax import lax
from jax.experimental import pallas as pl
from jax.experimental.pallas import tpu as pltpu
from jax.experimental.pallas import tpu_sc as plsc

N_HEAD = 4
N_NODES = 10000
N_EDGES = 320000
D = 128

NPAD = 10240
NT = 16
NC = 2
EPT = N_EDGES // NT
CH = 64
NK = 312
TAIL = 32
RPT = NPAD // NT
BN = 400
BF = 512


def _scores_body(x_ref, cs_ref, cd_ref, os_ref, od_ref):
    xb = x_ref[...]
    os_ref[...] = jnp.dot(xb, cs_ref[...], preferred_element_type=jnp.float32)
    od_ref[...] = jnp.dot(xb, cd_ref[...], preferred_element_type=jnp.float32)


def _tc_scores(x, csrc, cdst):
    return pl.pallas_call(
        _scores_body,
        grid=(N_NODES // BN,),
        in_specs=[
            pl.BlockSpec((BN, D), lambda i: (i, 0)),
            pl.BlockSpec((D, N_HEAD), lambda i: (0, 0)),
            pl.BlockSpec((D, N_HEAD), lambda i: (0, 0)),
        ],
        out_specs=[
            pl.BlockSpec((BN, N_HEAD), lambda i: (i, 0)),
            pl.BlockSpec((BN, N_HEAD), lambda i: (i, 0)),
        ],
        out_shape=[
            jax.ShapeDtypeStruct((N_NODES, N_HEAD), jnp.float32),
            jax.ShapeDtypeStruct((N_NODES, N_HEAD), jnp.float32),
        ],
    )(x, csrc, cdst)


def _sc_body(x_hbm, src_hbm, dst_hbm, pk_hbm, p_out, rsp_out,
             P_sh, pk_v, s0_v, d0_v, s1_v, d1_v, s2_v, d2_v, st_v, dt_v,
             rows0, rows1, rows2, rs_v,
             semi0, semi1, semi2, semg0, semg1, semg2,
             sems0, sems1, sems2):
    c = lax.axis_index("c")
    s = lax.axis_index("s")
    ebase = s * EPT
    rbase = s * RPT
    zeros16 = jnp.zeros((16,), jnp.float32)
    zeros16i = jnp.zeros((16,), jnp.int32)
    sv = (s0_v, s1_v, s2_v)
    dv = (d0_v, d1_v, d2_v)
    rows = (rows0, rows1, rows2)
    semi = (semi0, semi1, semi2)
    semg = (semg0, semg1, semg2)
    sems = (sems0, sems1, sems2)

    def _compute(svb, dvb, rb, ng):
        for g in range(ng):
            si = svb[pl.ds(g * 16, 16)]
            di = dvb[pl.ds(g * 16, 16)]
            gs = plsc.load_gather(pk_v, [si])
            gd = plsc.load_gather(pk_v, [di])
            z = (plsc.bitcast(gs << 16, jnp.float32)
                 + plsc.bitcast(gd & jnp.int32(-65536), jnp.float32))
            ee = jnp.exp(jnp.where(z >= 0, -z, -0.2 * z))
            plsc.addupdate_scatter(rs_v, [si], ee)
            for lane in range(16):
                e = g * 16 + lane
                bc = ee.at[jnp.full((16,), lane, jnp.int32)].get(
                    mode="promise_in_bounds")
                for kk in range(D // 16):
                    rb[e, pl.ds(kk * 16, 16)] = (
                        rb[e, pl.ds(kk * 16, 16)] * bc)

    for p in range(2):
        head = p * NC + c

        def _zero_rows(i, _):
            for kk in range(D // 16):
                rows0[i, pl.ds(kk * 16, 16)] = zeros16
                rows1[i, pl.ds(kk * 16, 16)] = zeros16
                rows2[i, pl.ds(kk * 16, 16)] = zeros16
            return 0

        lax.fori_loop(0, CH, _zero_rows, 0)

        def _zero_idx(i, _):
            s2_v[pl.ds(i * 16, 16)] = zeros16i
            return 0

        lax.fori_loop(0, CH // 16, _zero_idx, 0)
        for rep in range(RPT // CH):
            pltpu.sync_copy(rows0, P_sh.at[pl.ds(rbase + rep * CH, CH)])

        def _zero_rs(i, _):
            rs_v[pl.ds(i * 16, 16)] = zeros16
            return 0

        lax.fori_loop(0, NPAD // 16, _zero_rs, 0)

        pltpu.sync_copy(pk_hbm.at[pl.ds(head * NPAD, NPAD)], pk_v)
        plsc.subcore_barrier()

        pltpu.sync_copy(src_hbm.at[pl.ds(ebase, CH)], s0_v)
        pltpu.sync_copy(dst_hbm.at[pl.ds(ebase, CH)], d0_v)
        pltpu.async_copy(src_hbm.at[pl.ds(ebase + CH, CH)], s1_v, semi1)
        pltpu.async_copy(dst_hbm.at[pl.ds(ebase + CH, CH)], d1_v, semi1)
        pltpu.async_copy(x_hbm.at[d0_v], rows0, semg0)
        pltpu.async_copy(rows2, P_sh.at[s2_v], sems2, add=True)

        def _body(t, _):
            for u in range(3):
                k = 3 * t + u
                u1 = (u + 1) % 3
                u2 = (u + 2) % 3
                pltpu.make_async_copy(
                    src_hbm.at[pl.ds(ebase + (k + 1) * CH, CH)],
                    sv[u1], semi[u1]).wait()
                pltpu.make_async_copy(
                    dst_hbm.at[pl.ds(ebase + (k + 1) * CH, CH)],
                    dv[u1], semi[u1]).wait()
                pltpu.async_copy(x_hbm.at[dv[u1]], rows[u1], semg[u1])
                pltpu.make_async_copy(x_hbm.at[dv[u]], rows[u],
                                      semg[u]).wait()
                _compute(sv[u], dv[u], rows[u], CH // 16)
                pltpu.async_copy(rows[u], P_sh.at[sv[u]], sems[u], add=True)
                pltpu.make_async_copy(rows[u2], P_sh.at[sv[u2]],
                                      sems[u2]).wait()
                pltpu.async_copy(
                    src_hbm.at[pl.ds(ebase + (k + 2) * CH, CH)],
                    sv[u2], semi[u2])
                pltpu.async_copy(
                    dst_hbm.at[pl.ds(ebase + (k + 2) * CH, CH)],
                    dv[u2], semi[u2])
            return 0

        lax.fori_loop(0, NK // 3, _body, 0)
        pltpu.make_async_copy(rows2, P_sh.at[s2_v], sems2).wait()
        pltpu.make_async_copy(x_hbm.at[d0_v], rows0, semg0).wait()
        pltpu.make_async_copy(
            src_hbm.at[pl.ds(ebase + (NK + 1) * CH, CH)], s1_v, semi1).wait()
        pltpu.make_async_copy(
            dst_hbm.at[pl.ds(ebase + (NK + 1) * CH, CH)], d1_v, semi1).wait()

        toff = ebase + NK * CH
        pltpu.sync_copy(src_hbm.at[pl.ds(toff, TAIL)], st_v)
        pltpu.sync_copy(dst_hbm.at[pl.ds(toff, TAIL)], dt_v)
        pltpu.async_copy(x_hbm.at[dt_v], rows0.at[pl.ds(0, TAIL)],
                         semg0).wait()
        _compute(st_v, dt_v, rows0, TAIL // 16)
        pltpu.sync_copy(rows0.at[pl.ds(0, TAIL)], P_sh.at[st_v], add=True)
        plsc.subcore_barrier()

        pltpu.sync_copy(
            rs_v, rsp_out.at[pl.ds((head * NT + s) * NPAD, NPAD)])
        pltpu.sync_copy(P_sh.at[pl.ds(rbase, RPT)],
                        p_out.at[pl.ds(head * NPAD + rbase, RPT)])
        plsc.subcore_barrier()


def _sc_main(x, src, dst, pk):
    mesh = plsc.VectorSubcoreMesh(core_axis_name="c", subcore_axis_name="s")
    kern = functools.partial(
        pl.kernel,
        mesh=mesh,
        compiler_params=pltpu.CompilerParams(needs_layout_passes=False),
        out_type=[
            jax.ShapeDtypeStruct((N_HEAD * NPAD, D), jnp.float32),
            jax.ShapeDtypeStruct((N_HEAD * NT * NPAD,), jnp.float32),
        ],
        scratch_types=[
            pltpu.VMEM_SHARED((NPAD, D), jnp.float32),
            pltpu.VMEM((NPAD,), jnp.int32),
            pltpu.VMEM((CH,), jnp.int32),
            pltpu.VMEM((CH,), jnp.int32),
            pltpu.VMEM((CH,), jnp.int32),
            pltpu.VMEM((CH,), jnp.int32),
            pltpu.VMEM((CH,), jnp.int32),
            pltpu.VMEM((CH,), jnp.int32),
            pltpu.VMEM((TAIL,), jnp.int32),
            pltpu.VMEM((TAIL,), jnp.int32),
            pltpu.VMEM((CH, D), jnp.float32),
            pltpu.VMEM((CH, D), jnp.float32),
            pltpu.VMEM((CH, D), jnp.float32),
            pltpu.VMEM((NPAD,), jnp.float32),
            pltpu.SemaphoreType.DMA,
            pltpu.SemaphoreType.DMA,
            pltpu.SemaphoreType.DMA,
            pltpu.SemaphoreType.DMA,
            pltpu.SemaphoreType.DMA,
            pltpu.SemaphoreType.DMA,
            pltpu.SemaphoreType.DMA,
            pltpu.SemaphoreType.DMA,
            pltpu.SemaphoreType.DMA,
        ],
    )(_sc_body)
    return kern(x, src, dst, pk)


def _finish_body(p_ref, rsp_ref, w_ref, o_ref):
    rs = jnp.sum(rsp_ref[...], axis=1).reshape(1, BF, 1)
    o_ref[...] = p_ref[...] * w_ref[...] / rs


def _tc_finish(p_acc, rsp, wv3):
    return pl.pallas_call(
        _finish_body,
        grid=(N_HEAD, NPAD // BF),
        in_specs=[
            pl.BlockSpec((1, BF, D), lambda h, i: (h, i, 0)),
            pl.BlockSpec((1, NT, BF), lambda h, i: (h, 0, i)),
            pl.BlockSpec((1, 1, D), lambda h, i: (h, 0, 0)),
        ],
        out_specs=pl.BlockSpec((1, BF, D), lambda h, i: (h, i, 0)),
        out_shape=jax.ShapeDtypeStruct((N_HEAD, NPAD, D), jnp.float32),
    )(p_acc, rsp, wv3)


def kernel(x, edge_index, w, attn):
    src = edge_index[0]
    dst = edge_index[1]
    wv = w[:, 0, :]
    csrc = (wv * attn[:, :D, 0]).T
    cdst = (wv * attn[:, D:, 0]).T
    asrc, adst = _tc_scores(x, csrc, cdst)
    bs = lax.bitcast_convert_type(
        asrc.astype(jnp.bfloat16), jnp.uint16).astype(jnp.uint32)
    bd = lax.bitcast_convert_type(
        adst.astype(jnp.bfloat16), jnp.uint16).astype(jnp.uint32)
    pk = (bs | (bd << 16)).astype(jnp.int32).T
    pk = jnp.pad(pk, ((0, 0), (0, NPAD - N_NODES))).reshape(-1)
    src = jnp.pad(src, (0, 2 * CH))
    dst = jnp.pad(dst, (0, 2 * CH))
    p_acc, rsp = _sc_main(x, src, dst, pk)
    out = _tc_finish(p_acc.reshape(N_HEAD, NPAD, D),
                     rsp.reshape(N_HEAD, NT, NPAD),
                     wv.reshape(N_HEAD, 1, D))
    return out[:, :N_NODES, :]


# speedup vs baseline: 8.9043x; 1.1487x over previous
import functools

import jax
import jax.numpy as jnp
from jax import lax
from jax.experimental import pallas as pl
from jax.experimental.pallas import tpu as pltpu
from jax.experimental.pallas import tpu_sc as plsc

N_HEAD = 4
N_NODES = 10000
N_EDGES = 320000
D = 128

NPAD = 10240
NT = 16
NC = 2
EPT = N_EDGES // NT
CH = 64
NK = 312
TAIL = 32
RPT = NPAD // NT
BN = 400
BF = 512


def _scores_body(x_ref, cs_ref, cd_ref, os_ref, od_ref):
    xb = x_ref[...]
    os_ref[...] = jnp.dot(xb, cs_ref[...], preferred_element_type=jnp.float32)
    od_ref[...] = jnp.dot(xb, cd_ref[...], preferred_element_type=jnp.float32)


def _tc_scores(x, csrc, cdst):
    return pl.pallas_call(
        _scores_body,
        grid=(N_NODES // BN,),
        in_specs=[
            pl.BlockSpec((BN, D), lambda i: (i, 0)),
            pl.BlockSpec((D, N_HEAD), lambda i: (0, 0)),
            pl.BlockSpec((D, N_HEAD), lambda i: (0, 0)),
        ],
        out_specs=[
            pl.BlockSpec((BN, N_HEAD), lambda i: (i, 0)),
            pl.BlockSpec((BN, N_HEAD), lambda i: (i, 0)),
        ],
        out_shape=[
            jax.ShapeDtypeStruct((N_NODES, N_HEAD), jnp.float32),
            jax.ShapeDtypeStruct((N_NODES, N_HEAD), jnp.float32),
        ],
    )(x, csrc, cdst)


def _sc_body(x_hbm, src_hbm, dst_hbm, pk_hbm, p_out, rsp_out,
             P_sh, pk_v, s0_v, d0_v, s1_v, d1_v, s2_v, d2_v, st_v, dt_v,
             rows0, rows1, rows2, rs_v,
             semi0, semi1, semi2, semg0, semg1, semg2,
             sems0, sems1, sems2):
    c = lax.axis_index("c")
    s = lax.axis_index("s")
    ebase = s * EPT
    rbase = s * RPT
    zeros16 = jnp.zeros((16,), jnp.float32)
    zeros16i = jnp.zeros((16,), jnp.int32)
    sv = (s0_v, s1_v, s2_v)
    dv = (d0_v, d1_v, d2_v)
    rows = (rows0, rows1, rows2)
    semi = (semi0, semi1, semi2)
    semg = (semg0, semg1, semg2)
    sems = (sems0, sems1, sems2)

    def _compute(svb, dvb, rb, ng):
        for g in range(ng):
            si = svb[pl.ds(g * 16, 16)]
            di = dvb[pl.ds(g * 16, 16)]
            gs = plsc.load_gather(pk_v, [si])
            gd = plsc.load_gather(pk_v, [di])
            z = (plsc.bitcast(gs << 16, jnp.float32)
                 + plsc.bitcast(gd & jnp.int32(-65536), jnp.float32))
            ee = jnp.exp(jnp.where(z >= 0, -z, -0.2 * z))
            plsc.addupdate_scatter(rs_v, [si], ee)
            for lane in range(16):
                e = g * 16 + lane
                bc = ee.at[jnp.full((16,), lane, jnp.int32)].get(
                    mode="promise_in_bounds")
                for kk in range(D // 16):
                    rb[e, pl.ds(kk * 16, 16)] = (
                        rb[e, pl.ds(kk * 16, 16)] * bc)

    for p in range(2):
        head = p * NC + c

        def _zero_rows(i, _):
            for kk in range(D // 16):
                rows0[i, pl.ds(kk * 16, 16)] = zeros16
                rows1[i, pl.ds(kk * 16, 16)] = zeros16
                rows2[i, pl.ds(kk * 16, 16)] = zeros16
            return 0

        lax.fori_loop(0, CH, _zero_rows, 0)

        def _zero_idx(i, _):
            s2_v[pl.ds(i * 16, 16)] = zeros16i
            return 0

        lax.fori_loop(0, CH // 16, _zero_idx, 0)
        for rep in range(RPT // CH):
            pltpu.sync_copy(rows0, P_sh.at[pl.ds(rbase + rep * CH, CH)])

        def _zero_rs(i, _):
            rs_v[pl.ds(i * 16, 16)] = zeros16
            return 0

        lax.fori_loop(0, NPAD // 16, _zero_rs, 0)

        pltpu.sync_copy(pk_hbm.at[pl.ds(head * NPAD, NPAD)], pk_v)
        plsc.subcore_barrier()

        pltpu.sync_copy(src_hbm.at[pl.ds(ebase, CH)], s0_v)
        pltpu.sync_copy(dst_hbm.at[pl.ds(ebase, CH)], d0_v)
        pltpu.async_copy(src_hbm.at[pl.ds(ebase + CH, CH)], s1_v, semi1)
        pltpu.async_copy(dst_hbm.at[pl.ds(ebase + CH, CH)], d1_v, semi1)
        pltpu.async_copy(x_hbm.at[d0_v], rows0, semg0)
        pltpu.async_copy(rows2, P_sh.at[s2_v], sems2, add=True)

        def _body(t, _):
            for u in range(3):
                k = 3 * t + u
                u1 = (u + 1) % 3
                u2 = (u + 2) % 3
                pltpu.make_async_copy(
                    src_hbm.at[pl.ds(ebase + (k + 1) * CH, CH)],
                    sv[u1], semi[u1]).wait()
                pltpu.make_async_copy(
                    dst_hbm.at[pl.ds(ebase + (k + 1) * CH, CH)],
                    dv[u1], semi[u1]).wait()
                pltpu.async_copy(x_hbm.at[dv[u1]], rows[u1], semg[u1])
                pltpu.make_async_copy(rows[u2], P_sh.at[sv[u2]],
                                      sems[u2]).wait()
                pltpu.async_copy(
                    src_hbm.at[pl.ds(ebase + (k + 2) * CH, CH)],
                    sv[u2], semi[u2])
                pltpu.async_copy(
                    dst_hbm.at[pl.ds(ebase + (k + 2) * CH, CH)],
                    dv[u2], semi[u2])
                pltpu.make_async_copy(x_hbm.at[dv[u]], rows[u],
                                      semg[u]).wait()
                _compute(sv[u], dv[u], rows[u], CH // 16)
                pltpu.async_copy(rows[u], P_sh.at[sv[u]], sems[u], add=True)
            return 0

        lax.fori_loop(0, NK // 3, _body, 0)
        pltpu.make_async_copy(rows2, P_sh.at[s2_v], sems2).wait()
        pltpu.make_async_copy(x_hbm.at[d0_v], rows0, semg0).wait()
        pltpu.make_async_copy(
            src_hbm.at[pl.ds(ebase + (NK + 1) * CH, CH)], s1_v, semi1).wait()
        pltpu.make_async_copy(
            dst_hbm.at[pl.ds(ebase + (NK + 1) * CH, CH)], d1_v, semi1).wait()

        toff = ebase + NK * CH
        pltpu.sync_copy(src_hbm.at[pl.ds(toff, TAIL)], st_v)
        pltpu.sync_copy(dst_hbm.at[pl.ds(toff, TAIL)], dt_v)
        pltpu.async_copy(x_hbm.at[dt_v], rows0.at[pl.ds(0, TAIL)],
                         semg0).wait()
        _compute(st_v, dt_v, rows0, TAIL // 16)
        pltpu.sync_copy(rows0.at[pl.ds(0, TAIL)], P_sh.at[st_v], add=True)
        plsc.subcore_barrier()

        pltpu.sync_copy(
            rs_v, rsp_out.at[pl.ds((head * NT + s) * NPAD, NPAD)])
        pltpu.sync_copy(P_sh.at[pl.ds(rbase, RPT)],
                        p_out.at[pl.ds(head * NPAD + rbase, RPT)])
        plsc.subcore_barrier()


def _sc_main(x, src, dst, pk):
    mesh = plsc.VectorSubcoreMesh(core_axis_name="c", subcore_axis_name="s")
    kern = functools.partial(
        pl.kernel,
        mesh=mesh,
        compiler_params=pltpu.CompilerParams(needs_layout_passes=False),
        out_type=[
            jax.ShapeDtypeStruct((N_HEAD * NPAD, D), jnp.float32),
            jax.ShapeDtypeStruct((N_HEAD * NT * NPAD,), jnp.float32),
        ],
        scratch_types=[
            pltpu.VMEM_SHARED((NPAD, D), jnp.float32),
            pltpu.VMEM((NPAD,), jnp.int32),
            pltpu.VMEM((CH,), jnp.int32),
            pltpu.VMEM((CH,), jnp.int32),
            pltpu.VMEM((CH,), jnp.int32),
            pltpu.VMEM((CH,), jnp.int32),
            pltpu.VMEM((CH,), jnp.int32),
            pltpu.VMEM((CH,), jnp.int32),
            pltpu.VMEM((TAIL,), jnp.int32),
            pltpu.VMEM((TAIL,), jnp.int32),
            pltpu.VMEM((CH, D), jnp.float32),
            pltpu.VMEM((CH, D), jnp.float32),
            pltpu.VMEM((CH, D), jnp.float32),
            pltpu.VMEM((NPAD,), jnp.float32),
            pltpu.SemaphoreType.DMA,
            pltpu.SemaphoreType.DMA,
            pltpu.SemaphoreType.DMA,
            pltpu.SemaphoreType.DMA,
            pltpu.SemaphoreType.DMA,
            pltpu.SemaphoreType.DMA,
            pltpu.SemaphoreType.DMA,
            pltpu.SemaphoreType.DMA,
            pltpu.SemaphoreType.DMA,
        ],
    )(_sc_body)
    return kern(x, src, dst, pk)


def _finish_body(p_ref, rsp_ref, w_ref, o_ref):
    rs = jnp.sum(rsp_ref[...], axis=1).reshape(1, BF, 1)
    o_ref[...] = p_ref[...] * w_ref[...] / rs


def _tc_finish(p_acc, rsp, wv3):
    return pl.pallas_call(
        _finish_body,
        grid=(N_HEAD, NPAD // BF),
        in_specs=[
            pl.BlockSpec((1, BF, D), lambda h, i: (h, i, 0)),
            pl.BlockSpec((1, NT, BF), lambda h, i: (h, 0, i)),
            pl.BlockSpec((1, 1, D), lambda h, i: (h, 0, 0)),
        ],
        out_specs=pl.BlockSpec((1, BF, D), lambda h, i: (h, i, 0)),
        out_shape=jax.ShapeDtypeStruct((N_HEAD, NPAD, D), jnp.float32),
    )(p_acc, rsp, wv3)


def kernel(x, edge_index, w, attn):
    src = edge_index[0]
    dst = edge_index[1]
    wv = w[:, 0, :]
    csrc = (wv * attn[:, :D, 0]).T
    cdst = (wv * attn[:, D:, 0]).T
    asrc, adst = _tc_scores(x, csrc, cdst)
    bs = lax.bitcast_convert_type(
        asrc.astype(jnp.bfloat16), jnp.uint16).astype(jnp.uint32)
    bd = lax.bitcast_convert_type(
        adst.astype(jnp.bfloat16), jnp.uint16).astype(jnp.uint32)
    pk = (bs | (bd << 16)).astype(jnp.int32).T
    pk = jnp.pad(pk, ((0, 0), (0, NPAD - N_NODES))).reshape(-1)
    src = jnp.pad(src, (0, 2 * CH))
    dst = jnp.pad(dst, (0, 2 * CH))
    p_acc, rsp = _sc_main(x, src, dst, pk)
    out = _tc_finish(p_acc.reshape(N_HEAD, NPAD, D),
                     rsp.reshape(N_HEAD, NT, NPAD),
                     wv.reshape(N_HEAD, 1, D))
    return out[:, :N_NODES, :]


# speedup vs baseline: 9.1885x; 1.0319x over previous
import functools

import jax
import jax.numpy as jnp
from jax import lax
from jax.experimental import pallas as pl
from jax.experimental.pallas import tpu as pltpu
from jax.experimental.pallas import tpu_sc as plsc

N_HEAD = 4
N_NODES = 10000
N_EDGES = 320000
D = 128

NPAD = 10240
NT = 16
NC = 2
EPT = N_EDGES // NT
CH = 64
NK = 312
TAIL = 32
RPT = NPAD // NT
BN = 400
BF = 512


def _scores_body(x_ref, cs_ref, cd_ref, os_ref, od_ref):
    xb = x_ref[...]
    os_ref[...] = jnp.dot(xb, cs_ref[...], preferred_element_type=jnp.float32)
    od_ref[...] = jnp.dot(xb, cd_ref[...], preferred_element_type=jnp.float32)


def _tc_scores(x, csrc, cdst):
    return pl.pallas_call(
        _scores_body,
        grid=(N_NODES // BN,),
        in_specs=[
            pl.BlockSpec((BN, D), lambda i: (i, 0)),
            pl.BlockSpec((D, N_HEAD), lambda i: (0, 0)),
            pl.BlockSpec((D, N_HEAD), lambda i: (0, 0)),
        ],
        out_specs=[
            pl.BlockSpec((BN, N_HEAD), lambda i: (i, 0)),
            pl.BlockSpec((BN, N_HEAD), lambda i: (i, 0)),
        ],
        out_shape=[
            jax.ShapeDtypeStruct((N_NODES, N_HEAD), jnp.float32),
            jax.ShapeDtypeStruct((N_NODES, N_HEAD), jnp.float32),
        ],
    )(x, csrc, cdst)


def _sc_body(x_hbm, src_hbm, dst_hbm, pk_hbm, p_out, rsp_out,
             P_sh, pk_v, s0_v, d0_v, s1_v, d1_v, s2_v, d2_v, st_v, dt_v,
             rows0, rows1, rows2, rs_v,
             semi0, semi1, semi2, semg0, semg1, semg2,
             sems0, sems1, sems2):
    c = lax.axis_index("c")
    s = lax.axis_index("s")
    ebase = s * EPT
    rbase = s * RPT
    zeros16 = jnp.zeros((16,), jnp.float32)
    zeros16i = jnp.zeros((16,), jnp.int32)
    sv = (s0_v, s1_v, s2_v)
    dv = (d0_v, d1_v, d2_v)
    rows = (rows0, rows1, rows2)
    semi = (semi0, semi1, semi2)
    semg = (semg0, semg1, semg2)
    sems = (sems0, sems1, sems2)

    def _compute(svb, dvb, rb, g0, ng):
        for g in range(g0, ng):
            si = svb[pl.ds(g * 16, 16)]
            di = dvb[pl.ds(g * 16, 16)]
            gs = plsc.load_gather(pk_v, [si])
            gd = plsc.load_gather(pk_v, [di])
            z = (plsc.bitcast(gs << 16, jnp.float32)
                 + plsc.bitcast(gd & jnp.int32(-65536), jnp.float32))
            ee = jnp.exp(jnp.where(z >= 0, -z, -0.2 * z))
            plsc.addupdate_scatter(rs_v, [si], ee)
            for lane in range(16):
                e = g * 16 + lane
                bc = ee.at[jnp.full((16,), lane, jnp.int32)].get(
                    mode="promise_in_bounds")
                for kk in range(D // 16):
                    rb[e, pl.ds(kk * 16, 16)] = (
                        rb[e, pl.ds(kk * 16, 16)] * bc)

    for p in range(2):
        head = p * NC + c

        def _zero_rows(i, _):
            for kk in range(D // 16):
                rows0[i, pl.ds(kk * 16, 16)] = zeros16
                rows1[i, pl.ds(kk * 16, 16)] = zeros16
                rows2[i, pl.ds(kk * 16, 16)] = zeros16
            return 0

        lax.fori_loop(0, CH, _zero_rows, 0)

        def _zero_idx(i, _):
            s2_v[pl.ds(i * 16, 16)] = zeros16i
            return 0

        lax.fori_loop(0, CH // 16, _zero_idx, 0)
        for rep in range(RPT // CH):
            pltpu.sync_copy(rows0, P_sh.at[pl.ds(rbase + rep * CH, CH)])

        def _zero_rs(i, _):
            rs_v[pl.ds(i * 16, 16)] = zeros16
            return 0

        lax.fori_loop(0, NPAD // 16, _zero_rs, 0)

        pltpu.sync_copy(pk_hbm.at[pl.ds(head * NPAD, NPAD)], pk_v)
        plsc.subcore_barrier()

        pltpu.sync_copy(src_hbm.at[pl.ds(ebase, CH)], s0_v)
        pltpu.sync_copy(dst_hbm.at[pl.ds(ebase, CH)], d0_v)
        pltpu.async_copy(src_hbm.at[pl.ds(ebase + CH, CH)], s1_v, semi1)
        pltpu.async_copy(dst_hbm.at[pl.ds(ebase + CH, CH)], d1_v, semi1)
        pltpu.async_copy(x_hbm.at[d0_v], rows0, semg0)
        pltpu.async_copy(rows2, P_sh.at[s2_v], sems2, add=True)

        def _body(t, _):
            for u in range(3):
                k = 3 * t + u
                u1 = (u + 1) % 3
                u2 = (u + 2) % 3
                pltpu.make_async_copy(
                    src_hbm.at[pl.ds(ebase + (k + 1) * CH, CH)],
                    sv[u1], semi[u1]).wait()
                pltpu.make_async_copy(
                    dst_hbm.at[pl.ds(ebase + (k + 1) * CH, CH)],
                    dv[u1], semi[u1]).wait()
                pltpu.async_copy(x_hbm.at[dv[u1]], rows[u1], semg[u1])
                pltpu.make_async_copy(x_hbm.at[dv[u]], rows[u],
                                      semg[u]).wait()
                _compute(sv[u], dv[u], rows[u], 0, CH // 32)
                pltpu.make_async_copy(rows[u2], P_sh.at[sv[u2]],
                                      sems[u2]).wait()
                pltpu.async_copy(
                    src_hbm.at[pl.ds(ebase + (k + 2) * CH, CH)],
                    sv[u2], semi[u2])
                pltpu.async_copy(
                    dst_hbm.at[pl.ds(ebase + (k + 2) * CH, CH)],
                    dv[u2], semi[u2])
                _compute(sv[u], dv[u], rows[u], CH // 32, CH // 16)
                pltpu.async_copy(rows[u], P_sh.at[sv[u]], sems[u], add=True)
            return 0

        lax.fori_loop(0, NK // 3, _body, 0)
        pltpu.make_async_copy(rows2, P_sh.at[s2_v], sems2).wait()
        pltpu.make_async_copy(x_hbm.at[d0_v], rows0, semg0).wait()
        pltpu.make_async_copy(
            src_hbm.at[pl.ds(ebase + (NK + 1) * CH, CH)], s1_v, semi1).wait()
        pltpu.make_async_copy(
            dst_hbm.at[pl.ds(ebase + (NK + 1) * CH, CH)], d1_v, semi1).wait()

        toff = ebase + NK * CH
        pltpu.sync_copy(src_hbm.at[pl.ds(toff, TAIL)], st_v)
        pltpu.sync_copy(dst_hbm.at[pl.ds(toff, TAIL)], dt_v)
        pltpu.async_copy(x_hbm.at[dt_v], rows0.at[pl.ds(0, TAIL)],
                         semg0).wait()
        _compute(st_v, dt_v, rows0, 0, TAIL // 16)
        pltpu.sync_copy(rows0.at[pl.ds(0, TAIL)], P_sh.at[st_v], add=True)
        plsc.subcore_barrier()

        pltpu.sync_copy(
            rs_v, rsp_out.at[pl.ds((head * NT + s) * NPAD, NPAD)])
        pltpu.sync_copy(P_sh.at[pl.ds(rbase, RPT)],
                        p_out.at[pl.ds(head * NPAD + rbase, RPT)])
        plsc.subcore_barrier()


def _sc_main(x, src, dst, pk):
    mesh = plsc.VectorSubcoreMesh(core_axis_name="c", subcore_axis_name="s")
    kern = functools.partial(
        pl.kernel,
        mesh=mesh,
        compiler_params=pltpu.CompilerParams(needs_layout_passes=False),
        out_type=[
            jax.ShapeDtypeStruct((N_HEAD * NPAD, D), jnp.float32),
            jax.ShapeDtypeStruct((N_HEAD * NT * NPAD,), jnp.float32),
        ],
        scratch_types=[
            pltpu.VMEM_SHARED((NPAD, D), jnp.float32),
            pltpu.VMEM((NPAD,), jnp.int32),
            pltpu.VMEM((CH,), jnp.int32),
            pltpu.VMEM((CH,), jnp.int32),
            pltpu.VMEM((CH,), jnp.int32),
            pltpu.VMEM((CH,), jnp.int32),
            pltpu.VMEM((CH,), jnp.int32),
            pltpu.VMEM((CH,), jnp.int32),
            pltpu.VMEM((TAIL,), jnp.int32),
            pltpu.VMEM((TAIL,), jnp.int32),
            pltpu.VMEM((CH, D), jnp.float32),
            pltpu.VMEM((CH, D), jnp.float32),
            pltpu.VMEM((CH, D), jnp.float32),
            pltpu.VMEM((NPAD,), jnp.float32),
            pltpu.SemaphoreType.DMA,
            pltpu.SemaphoreType.DMA,
            pltpu.SemaphoreType.DMA,
            pltpu.SemaphoreType.DMA,
            pltpu.SemaphoreType.DMA,
            pltpu.SemaphoreType.DMA,
            pltpu.SemaphoreType.DMA,
            pltpu.SemaphoreType.DMA,
            pltpu.SemaphoreType.DMA,
        ],
    )(_sc_body)
    return kern(x, src, dst, pk)


def _finish_body(p_ref, rsp_ref, w_ref, o_ref):
    rs = jnp.sum(rsp_ref[...], axis=1).reshape(1, BF, 1)
    o_ref[...] = p_ref[...] * w_ref[...] / rs


def _tc_finish(p_acc, rsp, wv3):
    return pl.pallas_call(
        _finish_body,
        grid=(N_HEAD, NPAD // BF),
        in_specs=[
            pl.BlockSpec((1, BF, D), lambda h, i: (h, i, 0)),
            pl.BlockSpec((1, NT, BF), lambda h, i: (h, 0, i)),
            pl.BlockSpec((1, 1, D), lambda h, i: (h, 0, 0)),
        ],
        out_specs=pl.BlockSpec((1, BF, D), lambda h, i: (h, i, 0)),
        out_shape=jax.ShapeDtypeStruct((N_HEAD, NPAD, D), jnp.float32),
    )(p_acc, rsp, wv3)


def kernel(x, edge_index, w, attn):
    src = edge_index[0]
    dst = edge_index[1]
    wv = w[:, 0, :]
    csrc = (wv * attn[:, :D, 0]).T
    cdst = (wv * attn[:, D:, 0]).T
    asrc, adst = _tc_scores(x, csrc, cdst)
    bs = lax.bitcast_convert_type(
        asrc.astype(jnp.bfloat16), jnp.uint16).astype(jnp.uint32)
    bd = lax.bitcast_convert_type(
        adst.astype(jnp.bfloat16), jnp.uint16).astype(jnp.uint32)
    pk = (bs | (bd << 16)).astype(jnp.int32).T
    pk = jnp.pad(pk, ((0, 0), (0, NPAD - N_NODES))).reshape(-1)
    src = jnp.pad(src, (0, 2 * CH))
    dst = jnp.pad(dst, (0, 2 * CH))
    p_acc, rsp = _sc_main(x, src, dst, pk)
    out = _tc_finish(p_acc.reshape(N_HEAD, NPAD, D),
                     rsp.reshape(N_HEAD, NT, NPAD),
                     wv.reshape(N_HEAD, 1, D))
    return out[:, :N_NODES, :]
